# baseline (speedup 1.0000x reference)
import functools

import jax
import jax.numpy as jnp
from jax import lax
from jax.experimental import pallas as pl
from jax.experimental.pallas import tpu as pltpu
from jax.experimental.pallas import tpu_sc as plsc

BATCH = 16384
EMBED_DIM = 64
NUM_CORES = 2
NUM_SUBCORES = 16
NUM_WORKERS = NUM_CORES * NUM_SUBCORES
BPW = BATCH // NUM_WORKERS
GRP = 128
NGRP = BPW // GRP
LANES = 16
NBUF = 2


def _dot_body(user_hbm, mission_hbm, utab_hbm, mtab_hbm, out_hbm,
              uidx, midx, ubuf, mbuf, out_v, sem):
    wid = lax.axis_index("s") * NUM_CORES + lax.axis_index("c")
    base = wid * BPW

    pltpu.sync_copy(user_hbm.at[pl.ds(base, BPW)], uidx)
    pltpu.sync_copy(mission_hbm.at[pl.ds(base, BPW)], midx)

    def fire(g, buf):
        sl = pl.ds(g * GRP, GRP)
        cps = []
        for d in range(EMBED_DIM):
            cps.append(pltpu.async_copy(
                utab_hbm.at[d].at[uidx.at[sl]], ubuf.at[buf, d], sem))
            cps.append(pltpu.async_copy(
                mtab_hbm.at[d].at[midx.at[sl]], mbuf.at[buf, d], sem))
        return cps

    def compute(g, buf):
        def body(d, accs):
            new = []
            for k in range(GRP // LANES):
                sl = pl.ds(k * LANES, LANES)
                new.append(accs[k] + ubuf[buf, d, sl] * mbuf[buf, d, sl])
            return tuple(new)

        accs = tuple(jnp.zeros((LANES,), jnp.float32)
                     for _ in range(GRP // LANES))
        accs = lax.fori_loop(0, EMBED_DIM, body, accs)
        for k in range(GRP // LANES):
            out_v[pl.ds(g * GRP + k * LANES, LANES)] = accs[k]

    pending = fire(0, 0)
    for g in range(NGRP):
        if g + 1 < NGRP:
            nxt = fire(g + 1, (g + 1) % NBUF)
        for cp in pending:
            cp.wait()
        compute(g, g % NBUF)
        if g + 1 < NGRP:
            pending = nxt

    pltpu.sync_copy(out_v, out_hbm.at[pl.ds(base, BPW)])


@functools.partial(jax.jit, static_argnames=())
def kernel(user, mission, user_table, mission_table):
    mesh = plsc.VectorSubcoreMesh(core_axis_name="c", subcore_axis_name="s")
    run = functools.partial(
        pl.kernel,
        mesh=mesh,
        compiler_params=pltpu.CompilerParams(
            needs_layout_passes=False, use_tc_tiling_on_sc=False),
        out_type=jax.ShapeDtypeStruct((BATCH,), jnp.float32),
        scratch_types=[
            pltpu.VMEM((BPW,), jnp.int32),
            pltpu.VMEM((BPW,), jnp.int32),
            pltpu.VMEM((NBUF, EMBED_DIM, GRP), jnp.float32),
            pltpu.VMEM((NBUF, EMBED_DIM, GRP), jnp.float32),
            pltpu.VMEM((BPW,), jnp.float32),
            pltpu.SemaphoreType.DMA,
        ],
    )(_dot_body)
    return run(user, mission, user_table.T, mission_table.T)


# speedup vs baseline: 10.7814x; 10.7814x over previous
import functools

import jax
import jax.numpy as jnp
from jax import lax
from jax.experimental import pallas as pl
from jax.experimental.pallas import tpu as pltpu
from jax.experimental.pallas import tpu_sc as plsc

BATCH = 16384
EMBED_DIM = 64
ROW = 2 * EMBED_DIM
NUM_CORES = 2
NUM_SUBCORES = 16
NUM_WORKERS = NUM_CORES * NUM_SUBCORES
BPW = BATCH // NUM_WORKERS
CHUNK = 128
NCHUNK = BPW // CHUNK
LANES = 16
NBUF = 2


def _dot_body(user_hbm, mission_hbm, tab_hbm, out_hbm,
              uidx, midx, ubuf, mbuf, out_v, sem):
    wid = lax.axis_index("s") * NUM_CORES + lax.axis_index("c")
    base = wid * BPW

    pltpu.sync_copy(user_hbm.at[pl.ds(base, BPW)], uidx)
    pltpu.sync_copy(mission_hbm.at[pl.ds(base, BPW)], midx)

    def fire(c, buf):
        sl = pl.ds(c * CHUNK, CHUNK)
        cp_u = pltpu.async_copy(tab_hbm.at[uidx.at[sl]], ubuf.at[buf], sem)
        cp_m = pltpu.async_copy(tab_hbm.at[midx.at[sl]], mbuf.at[buf], sem)
        return cp_u, cp_m

    def compute(c, buf):
        for g in range(CHUNK // LANES):
            rv = jnp.full((LANES,), g * LANES, jnp.int32) + lax.iota(
                jnp.int32, LANES)

            def body(d, acc):
                dv = jnp.full((LANES,), d, jnp.int32)
                u = plsc.load_gather(ubuf.at[buf], [rv, dv])
                m = plsc.load_gather(mbuf.at[buf], [rv, dv + EMBED_DIM])
                return acc + u * m

            acc = lax.fori_loop(0, EMBED_DIM, body,
                                jnp.zeros((LANES,), jnp.float32), unroll=8)
            out_v[pl.ds(c * CHUNK + g * LANES, LANES)] = acc

    pending = fire(0, 0)
    for c in range(NCHUNK):
        if c + 1 < NCHUNK:
            nxt = fire(c + 1, (c + 1) % NBUF)
        for cp in pending:
            cp.wait()
        compute(c, c % NBUF)
        if c + 1 < NCHUNK:
            pending = nxt

    pltpu.sync_copy(out_v, out_hbm.at[pl.ds(base, BPW)])


@functools.partial(jax.jit, static_argnames=())
def kernel(user, mission, user_table, mission_table):
    mesh = plsc.VectorSubcoreMesh(core_axis_name="c", subcore_axis_name="s")
    run = functools.partial(
        pl.kernel,
        mesh=mesh,
        compiler_params=pltpu.CompilerParams(needs_layout_passes=False),
        out_type=jax.ShapeDtypeStruct((BATCH,), jnp.float32),
        scratch_types=[
            pltpu.VMEM((BPW,), jnp.int32),
            pltpu.VMEM((BPW,), jnp.int32),
            pltpu.VMEM((NBUF, CHUNK, ROW), jnp.float32),
            pltpu.VMEM((NBUF, CHUNK, ROW), jnp.float32),
            pltpu.VMEM((BPW,), jnp.float32),
            pltpu.SemaphoreType.DMA,
        ],
    )(_dot_body)
    big = jnp.concatenate([user_table, mission_table], axis=1)
    return run(user, mission, big)
